# baseline (speedup 1.0000x reference)
import functools

import jax
import jax.numpy as jnp
from jax import lax
from jax.experimental import pallas as pl
from jax.experimental.pallas import tpu as pltpu
from jax.experimental.pallas import tpu_sc as plsc

VOCAB = 100000
HIDDEN = 64
WINDOW = 10
BATCH = 1024
K = 2 * WINDOW

NUM_CORES = 2
NUM_SUBCORES = 16
NW = NUM_CORES * NUM_SUBCORES
BPW = BATCH // NW
ROWS = BPW * K
CHUNK = 128
NCHUNK = ROWS // CHUNK
LANES = 16


@functools.cache
def _build_cbow_pool():
    mesh = plsc.VectorSubcoreMesh(
        core_axis_name="c", subcore_axis_name="s",
        num_cores=NUM_CORES, num_subcores=NUM_SUBCORES,
    )

    @functools.partial(
        pl.kernel,
        mesh=mesh,
        out_type=jax.ShapeDtypeStruct((BATCH, HIDDEN), jnp.float32),
        scratch_types=[
            pltpu.VMEM((NCHUNK, CHUNK), jnp.int32),
            pltpu.VMEM((ROWS, HIDDEN), jnp.float32),
            pltpu.VMEM((BPW, HIDDEN), jnp.float32),
            pltpu.SemaphoreType.DMA,
        ],
        compiler_params=pltpu.CompilerParams(use_tc_tiling_on_sc=False),
    )
    def _cbow_pool(ctx_hbm, table_hbm, h_hbm, idx_v, rows_v, h_v, sem):
        wid = lax.axis_index("s") * NUM_CORES + lax.axis_index("c")
        base = wid * BPW

        pltpu.sync_copy(ctx_hbm.at[wid], idx_v)

        copies = [
            pltpu.async_copy(
                table_hbm.at[idx_v.at[i]],
                rows_v.at[pl.ds(i * CHUNK, CHUNK)],
                sem,
            )
            for i in range(NCHUNK)
        ]
        for c in copies:
            c.wait()

        def body(b, carry):
            row0 = b * K
            for c in range(HIDDEN // LANES):
                sl = pl.ds(c * LANES, LANES)
                acc = rows_v[row0, sl]
                for j in range(1, K):
                    acc = acc + rows_v[row0 + j, sl]
                h_v[b, sl] = acc * (1.0 / K)
            return carry

        lax.fori_loop(0, BPW, body, 0)

        pltpu.sync_copy(h_v, h_hbm.at[pl.ds(base, BPW)])

    return _cbow_pool


_VB = 2048
_G = (VOCAB + _VB - 1) // _VB
_TAILR = VOCAB - (_G - 1) * _VB
_NBUF = 4


def _mm_body(h_ref, w_ref, o_hbm, o_buf, sems):
    j = pl.program_id(0)
    slot = lax.rem(j, _NBUF)

    @pl.when(j >= _NBUF)
    def _wait_prev():
        pltpu.make_async_copy(
            o_buf.at[slot], o_hbm.at[pl.ds(0, _VB)], sems.at[slot]
        ).wait()

    o_buf[slot] = lax.dot_general(
        w_ref[...], h_ref[...],
        (((0,), (1,)), ((), ())),
        preferred_element_type=jnp.float32,
    )

    @pl.when(j < _G - 1)
    def _fire():
        pltpu.make_async_copy(
            o_buf.at[slot], o_hbm.at[pl.ds(j * _VB, _VB)], sems.at[slot]
        ).start()

    @pl.when(j == _G - 1)
    def _tail_and_drain():
        pltpu.make_async_copy(
            o_buf.at[slot, pl.ds(0, _TAILR)],
            o_hbm.at[pl.ds((_G - 1) * _VB, _TAILR)], sems.at[slot],
        ).start()
        for step in range(_G - _NBUF, _G - 1):
            s = step % _NBUF
            pltpu.make_async_copy(
                o_buf.at[s], o_hbm.at[pl.ds(0, _VB)], sems.at[s]
            ).wait()
        pltpu.make_async_copy(
            o_buf.at[slot, pl.ds(0, _TAILR)],
            o_hbm.at[pl.ds(0, _TAILR)], sems.at[slot],
        ).wait()


def _project_t(h, W_out):
    return pl.pallas_call(
        _mm_body,
        grid=(_G,),
        in_specs=[
            pl.BlockSpec((BATCH, HIDDEN), lambda j: (0, 0)),
            pl.BlockSpec((HIDDEN, _VB), lambda j: (0, j)),
        ],
        out_specs=pl.BlockSpec(memory_space=pltpu.MemorySpace.HBM),
        out_shape=jax.ShapeDtypeStruct((VOCAB, BATCH), jnp.float32),
        scratch_shapes=[
            pltpu.VMEM((_NBUF, _VB, BATCH), jnp.float32),
            pltpu.SemaphoreType.DMA((_NBUF,)),
        ],
        compiler_params=pltpu.CompilerParams(
            dimension_semantics=("arbitrary",),
            vmem_limit_bytes=60 * 1024 * 1024,
        ),
    )(h, W_out)


def kernel(contexts, emb_table, W_out):
    ctx = contexts.astype(jnp.int32).reshape(NW, NCHUNK, CHUNK)
    h = _build_cbow_pool()(ctx, emb_table)
    return _project_t(h, W_out).T


# speedup vs baseline: 1.0065x; 1.0065x over previous
import functools

import jax
import jax.numpy as jnp
from jax import lax
from jax.experimental import pallas as pl
from jax.experimental.pallas import tpu as pltpu
from jax.experimental.pallas import tpu_sc as plsc

VOCAB = 100000
HIDDEN = 64
WINDOW = 10
BATCH = 1024
K = 2 * WINDOW

NUM_CORES = 2
NUM_SUBCORES = 16
NW = NUM_CORES * NUM_SUBCORES
BPW = BATCH // NW
ROWS = BPW * K
CHUNK = 128
NCHUNK = ROWS // CHUNK
LANES = 16


@functools.cache
def _build_cbow_pool():
    mesh = plsc.VectorSubcoreMesh(
        core_axis_name="c", subcore_axis_name="s",
        num_cores=NUM_CORES, num_subcores=NUM_SUBCORES,
    )

    @functools.partial(
        pl.kernel,
        mesh=mesh,
        out_type=jax.ShapeDtypeStruct((BATCH, HIDDEN), jnp.float32),
        scratch_types=[
            pltpu.VMEM((NCHUNK, CHUNK), jnp.int32),
            pltpu.VMEM((ROWS, HIDDEN), jnp.float32),
            pltpu.VMEM((BPW, HIDDEN), jnp.float32),
            pltpu.SemaphoreType.DMA,
        ],
        compiler_params=pltpu.CompilerParams(use_tc_tiling_on_sc=False),
    )
    def _cbow_pool(ctx_hbm, table_hbm, h_hbm, idx_v, rows_v, h_v, sem):
        wid = lax.axis_index("s") * NUM_CORES + lax.axis_index("c")
        base = wid * BPW

        pltpu.sync_copy(ctx_hbm.at[wid], idx_v)

        copies = [
            pltpu.async_copy(
                table_hbm.at[idx_v.at[i]],
                rows_v.at[pl.ds(i * CHUNK, CHUNK)],
                sem,
            )
            for i in range(NCHUNK)
        ]
        for c in copies:
            c.wait()

        def body(b, carry):
            row0 = b * K
            for c in range(HIDDEN // LANES):
                sl = pl.ds(c * LANES, LANES)
                acc = rows_v[row0, sl]
                for j in range(1, K):
                    acc = acc + rows_v[row0 + j, sl]
                h_v[b, sl] = acc * (1.0 / K)
            return carry

        lax.fori_loop(0, BPW, body, 0)

        pltpu.sync_copy(h_v, h_hbm.at[pl.ds(base, BPW)])

    return _cbow_pool


_VB = 4096


def _mm_body(h_ref, w_ref, o_ref):
    o_ref[...] = lax.dot_general(
        w_ref[...], h_ref[...],
        (((0,), (1,)), ((), ())),
        preferred_element_type=jnp.float32,
    )


def _project_t(h, W_out):
    grid = pl.cdiv(VOCAB, _VB)
    return pl.pallas_call(
        _mm_body,
        grid=(grid,),
        in_specs=[
            pl.BlockSpec((BATCH, HIDDEN), lambda j: (0, 0)),
            pl.BlockSpec((HIDDEN, _VB), lambda j: (0, j)),
        ],
        out_specs=pl.BlockSpec((_VB, BATCH), lambda j: (j, 0)),
        out_shape=jax.ShapeDtypeStruct((VOCAB, BATCH), jnp.float32),
        compiler_params=pltpu.CompilerParams(
            dimension_semantics=("arbitrary",),
        ),
    )(h, W_out)


def kernel(contexts, emb_table, W_out):
    ctx = contexts.astype(jnp.int32).reshape(NW, NCHUNK, CHUNK)
    h = _build_cbow_pool()(ctx, emb_table)
    return _project_t(h, W_out).T


# speedup vs baseline: 1.0095x; 1.0029x over previous
import functools

import jax
import jax.numpy as jnp
from jax import lax
from jax.experimental import pallas as pl
from jax.experimental.pallas import tpu as pltpu
from jax.experimental.pallas import tpu_sc as plsc

VOCAB = 100000
HIDDEN = 64
WINDOW = 10
BATCH = 1024
K = 2 * WINDOW

NUM_CORES = 2
NUM_SUBCORES = 16
NW = NUM_CORES * NUM_SUBCORES
BPW = BATCH // NW
ROWS = BPW * K
CHUNK = 128
NCHUNK = ROWS // CHUNK
LANES = 16


@functools.cache
def _build_cbow_pool():
    mesh = plsc.VectorSubcoreMesh(
        core_axis_name="c", subcore_axis_name="s",
        num_cores=NUM_CORES, num_subcores=NUM_SUBCORES,
    )

    @functools.partial(
        pl.kernel,
        mesh=mesh,
        out_type=jax.ShapeDtypeStruct((BATCH, HIDDEN), jnp.float32),
        scratch_types=[
            pltpu.VMEM((NCHUNK, CHUNK), jnp.int32),
            pltpu.VMEM((ROWS, HIDDEN), jnp.float32),
            pltpu.VMEM((BPW, HIDDEN), jnp.float32),
            pltpu.SemaphoreType.DMA,
        ],
        compiler_params=pltpu.CompilerParams(use_tc_tiling_on_sc=False, skip_device_barrier=True),
    )
    def _cbow_pool(ctx_hbm, table_hbm, h_hbm, idx_v, rows_v, h_v, sem):
        wid = lax.axis_index("s") * NUM_CORES + lax.axis_index("c")
        base = wid * BPW

        pltpu.sync_copy(ctx_hbm.at[wid], idx_v)

        copies = [
            pltpu.async_copy(
                table_hbm.at[idx_v.at[i]],
                rows_v.at[pl.ds(i * CHUNK, CHUNK)],
                sem,
            )
            for i in range(NCHUNK)
        ]
        for c in copies:
            c.wait()

        def body(b, carry):
            row0 = b * K
            for c in range(HIDDEN // LANES):
                sl = pl.ds(c * LANES, LANES)
                acc = rows_v[row0, sl]
                for j in range(1, K):
                    acc = acc + rows_v[row0 + j, sl]
                h_v[b, sl] = acc * (1.0 / K)
            return carry

        lax.fori_loop(0, BPW, body, 0)

        pltpu.sync_copy(h_v, h_hbm.at[pl.ds(base, BPW)])

    return _cbow_pool


_VB = 4096


def _mm_body(h_ref, w_ref, o_ref):
    o_ref[...] = lax.dot_general(
        w_ref[...], h_ref[...],
        (((0,), (1,)), ((), ())),
        preferred_element_type=jnp.float32,
    )


def _project_t(h, W_out):
    grid = pl.cdiv(VOCAB, _VB)
    return pl.pallas_call(
        _mm_body,
        grid=(grid,),
        in_specs=[
            pl.BlockSpec((BATCH, HIDDEN), lambda j: (0, 0)),
            pl.BlockSpec((HIDDEN, _VB), lambda j: (0, j)),
        ],
        out_specs=pl.BlockSpec((_VB, BATCH), lambda j: (j, 0)),
        out_shape=jax.ShapeDtypeStruct((VOCAB, BATCH), jnp.float32),
        compiler_params=pltpu.CompilerParams(
            dimension_semantics=("arbitrary",),
        ),
    )(h, W_out)


def kernel(contexts, emb_table, W_out):
    ctx = contexts.astype(jnp.int32).reshape(NW, NCHUNK, CHUNK)
    h = _build_cbow_pool()(ctx, emb_table)
    return _project_t(h, W_out).T
